# Initial kernel scaffold; baseline (speedup 1.0000x reference)
import jax
import jax.numpy as jnp
from jax.experimental import pallas as pl


def kernel(node_feats, edge_feats, edge_index, W_node, b_node, W_edge, b_edge, W_m1, b_m1, W_m2, b_m2, W_m3, b_m3):
    raise NotImplementedError("write your pallas kernel here")


# speedup vs baseline: 3.5844x; 3.5844x over previous
import functools

import jax
import jax.numpy as jnp
from jax import lax
from jax.experimental import pallas as pl
from jax.experimental.pallas import tpu as pltpu
from jax.experimental.pallas import tpu_sc as plsc

N = 10000
E = 320000
IN_FEATS = 128
F = 16

CHUNK = 128
NCH = E // CHUNK
NW = 32
CPW = NCH // NW
CREM = NCH - CPW * NW
G = 8
NROW = N // 16

@functools.cache
def _sc_mesh():
    return plsc.VectorSubcoreMesh(core_axis_name="c", subcore_axis_name="s")


def _worker_range(w):
    nc = jnp.where(w < CREM, CPW + 1, CPW)
    cbase = w * CPW + jnp.minimum(w, CREM)
    return cbase, nc


def _e0(cbase, j):
    return pl.multiple_of((cbase + j) * CHUNK, CHUNK)


def _gather_body(h0_hbm, src_hbm, dst_hbm, hs_hbm, hd_hbm,
                 idx_s, idx_d, rows_s, rows_d, sem_i, sem_g, sem_w):
    cid = lax.axis_index("c")
    sid = lax.axis_index("s")
    w = sid * 2 + cid
    cbase, nc = _worker_range(w)

    def chunk_grp(c0, gsz):
        pend = []
        for b in range(gsz):
            e0 = _e0(cbase, c0 + b)
            pend.append(pltpu.async_copy(src_hbm.at[pl.ds(e0, CHUNK)], idx_s.at[b], sem_i))
            pend.append(pltpu.async_copy(dst_hbm.at[pl.ds(e0, CHUNK)], idx_d.at[b], sem_i))
        for d in pend:
            d.wait()
        pend = []
        for b in range(gsz):
            pend.append(pltpu.async_copy(h0_hbm.at[idx_s.at[b]], rows_s.at[b], sem_g))
            pend.append(pltpu.async_copy(h0_hbm.at[idx_d.at[b]], rows_d.at[b], sem_g))
        for d in pend:
            d.wait()
        pend = []
        for b in range(gsz):
            e0 = _e0(cbase, c0 + b)
            pend.append(pltpu.async_copy(rows_s.at[b], hs_hbm.at[pl.ds(e0, CHUNK)], sem_w))
            pend.append(pltpu.async_copy(rows_d.at[b], hd_hbm.at[pl.ds(e0, CHUNK)], sem_w))
        for d in pend:
            d.wait()

    def grp(g, carry):
        chunk_grp(g * G, G)
        return carry

    lax.fori_loop(0, nc // G, grp, 0)

    def tail(j, carry):
        chunk_grp(j, 1)
        return carry

    lax.fori_loop((nc // G) * G, nc, tail, 0)


@functools.cache
def _gather():
    return pl.kernel(
        _gather_body,
        out_type=[jax.ShapeDtypeStruct((E, F), jnp.float32),
                  jax.ShapeDtypeStruct((E, F), jnp.float32)],
        mesh=_sc_mesh(),
        compiler_params=pltpu.CompilerParams(use_tc_tiling_on_sc=False),
        scratch_types=[
            pltpu.VMEM((G, CHUNK), jnp.int32),
            pltpu.VMEM((G, CHUNK), jnp.int32),
            pltpu.VMEM((G, CHUNK, F), jnp.float32),
            pltpu.VMEM((G, CHUNK, F), jnp.float32),
            pltpu.SemaphoreType.DMA,
            pltpu.SemaphoreType.DMA,
            pltpu.SemaphoreType.DMA,
        ],
    )


def _scatter_body(mail2_hbm, dst_hbm, zrow_hbm, out_hbm,
                  idx_d, mail_v, zer_v, acc, sem_l, sem_s):
    cid = lax.axis_index("c")
    sid = lax.axis_index("s")
    w = sid * 2 + cid
    cbase, nc = _worker_range(w)

    pltpu.sync_copy(zrow_hbm, zer_v)
    pltpu.sync_copy(zer_v, acc.at[pl.ds(sid * NROW, NROW)])
    plsc.subcore_barrier()

    def chunk_grp(c0, gsz):
        pend = []
        for b in range(gsz):
            e0 = _e0(cbase, c0 + b)
            pend.append(pltpu.async_copy(dst_hbm.at[pl.ds(e0, CHUNK)], idx_d.at[b], sem_l))
            pend.append(pltpu.async_copy(mail2_hbm.at[pl.ds(e0, CHUNK)], mail_v.at[b], sem_l))
        for d in pend:
            d.wait()
        pend = []
        for b in range(gsz):
            pend.append(pltpu.async_copy(mail_v.at[b], acc.at[idx_d.at[b]], sem_s, add=True))
        for d in pend:
            d.wait()

    def grp(g, carry):
        chunk_grp(g * G, G)
        return carry

    lax.fori_loop(0, nc // G, grp, 0)

    def tail(j, carry):
        chunk_grp(j, 1)
        return carry

    lax.fori_loop((nc // G) * G, nc, tail, 0)

    plsc.subcore_barrier()
    pltpu.sync_copy(acc.at[pl.ds(sid * NROW, NROW)],
                    out_hbm.at[pl.ds(cid * N + sid * NROW, NROW)])


@functools.cache
def _scatter():
    return pl.kernel(
        _scatter_body,
        out_type=jax.ShapeDtypeStruct((2 * N, 2 * F), jnp.float32),
        mesh=_sc_mesh(),
        compiler_params=pltpu.CompilerParams(use_tc_tiling_on_sc=False),
        scratch_types=[
            pltpu.VMEM((G, CHUNK), jnp.int32),
            pltpu.VMEM((G, CHUNK, 2 * F), jnp.float32),
            pltpu.VMEM((NROW, 2 * F), jnp.float32),
            pltpu.VMEM_SHARED((N, 2 * F), jnp.float32),
            pltpu.SemaphoreType.DMA,
            pltpu.SemaphoreType.DMA,
        ],
    )


def _node_body(x_ref, wnt_ref, bn_ref, out_ref):
    out_ref[...] = (jnp.dot(x_ref[...], wnt_ref[...],
                            preferred_element_type=jnp.float32) + bn_ref[...])


def _edge_body(hs_ref, hd_ref, ef_ref, wc_ref, bp_ref, s_ref, w2t_ref, b2_ref,
               out_ref):
    ef = ef_ref[...]
    wc = wc_ref[...]
    msg = jnp.dot(hs_ref[...], wc[0:F], preferred_element_type=jnp.float32)
    msg = msg + jnp.dot(hd_ref[...], wc[F:2 * F], preferred_element_type=jnp.float32)
    msg = msg + jnp.dot(ef, wc[2 * F:3 * F], preferred_element_type=jnp.float32)
    msg = msg + bp_ref[...]
    d = jnp.dot(ef, s_ref[...], preferred_element_type=jnp.float32)
    mail = jnp.dot(d * msg, w2t_ref[...], preferred_element_type=jnp.float32) + b2_ref[...]
    out_ref[:, 0:F] = mail
    out_ref[:, F:2 * F] = jnp.ones_like(mail)


def _final_body(p0_ref, p1_ref, h0_ref, w3t_ref, b3_ref, out_ref):
    t = p0_ref[...] + p1_ref[...]
    s = t[:, 0:F]
    deg = t[:, F:F + 1]
    h0b = h0_ref[...]
    h = jnp.where(deg > 0.0, s / jnp.maximum(deg, 1.0), h0b)
    h = jnp.maximum(h, 0.0)
    w3t = w3t_ref[...]
    out_ref[...] = (h0b + b3_ref[...]
                    + jnp.dot(h0b, w3t[0:F], preferred_element_type=jnp.float32)
                    + jnp.dot(h, w3t[F:2 * F], preferred_element_type=jnp.float32))


NBLK = 2000
EBLK = 2000


def _node_linear(node_feats, wnt, bn):
    return pl.pallas_call(
        _node_body,
        grid=(N // NBLK,),
        in_specs=[pl.BlockSpec((NBLK, IN_FEATS), lambda i: (i, 0)),
                  pl.BlockSpec((IN_FEATS, F), lambda i: (0, 0)),
                  pl.BlockSpec((1, F), lambda i: (0, 0))],
        out_specs=pl.BlockSpec((NBLK, F), lambda i: (i, 0)),
        out_shape=jax.ShapeDtypeStruct((N, F), jnp.float32),
    )(node_feats, wnt, bn)


def _edge_stage(hs, hd, ef, wc, bp, smat, w2t, b2):
    return pl.pallas_call(
        _edge_body,
        grid=(E // EBLK,),
        in_specs=[pl.BlockSpec((EBLK, F), lambda i: (i, 0)),
                  pl.BlockSpec((EBLK, F), lambda i: (i, 0)),
                  pl.BlockSpec((EBLK, F), lambda i: (i, 0)),
                  pl.BlockSpec((3 * F, F * F), lambda i: (0, 0)),
                  pl.BlockSpec((1, F * F), lambda i: (0, 0)),
                  pl.BlockSpec((F, F * F), lambda i: (0, 0)),
                  pl.BlockSpec((F * F, F), lambda i: (0, 0)),
                  pl.BlockSpec((1, F), lambda i: (0, 0))],
        out_specs=pl.BlockSpec((EBLK, 2 * F), lambda i: (i, 0)),
        out_shape=jax.ShapeDtypeStruct((E, 2 * F), jnp.float32),
    )(hs, hd, ef, wc, bp, smat, w2t, b2)


def _final_stage(pacc, h0, w3t, b3):
    nb = N // NBLK
    return pl.pallas_call(
        _final_body,
        grid=(nb,),
        in_specs=[pl.BlockSpec((NBLK, 2 * F), lambda i: (i, 0)),
                  pl.BlockSpec((NBLK, 2 * F), lambda i: (i + nb, 0)),
                  pl.BlockSpec((NBLK, F), lambda i: (i, 0)),
                  pl.BlockSpec((2 * F, F), lambda i: (0, 0)),
                  pl.BlockSpec((1, F), lambda i: (0, 0))],
        out_specs=pl.BlockSpec((NBLK, F), lambda i: (i, 0)),
        out_shape=jax.ShapeDtypeStruct((N, F), jnp.float32),
    )(pacc, pacc, h0, w3t, b3)


def kernel(node_feats, edge_feats, edge_index, W_node, b_node, W_edge, b_edge,
           W_m1, b_m1, W_m2, b_m2, W_m3, b_m3):
    a3 = W_m1[:, 2 * F:3 * F].T
    wc = jnp.concatenate([W_m1[:, 0:F].T, W_m1[:, F:2 * F].T,
                          W_edge.T @ a3], axis=0)
    bp = (b_m1 + b_edge @ a3).reshape(1, F * F)
    col = jax.lax.broadcasted_iota(jnp.int32, (F, F * F), 1) % F
    row = jax.lax.broadcasted_iota(jnp.int32, (F, F * F), 0)
    smat = (col == row).astype(jnp.float32)
    w2t = W_m2.T
    b2 = b_m2.reshape(1, F)
    w3t = W_m3.T
    b3 = b_m3.reshape(1, F)
    bn = b_node.reshape(1, F)
    wnt = W_node.T

    src = edge_index[0]
    dst = edge_index[1]
    zrow = jnp.zeros((NROW, 2 * F), jnp.float32)

    h0 = _node_linear(node_feats, wnt, bn)
    hs, hd = _gather()(h0, src, dst)
    mail2 = _edge_stage(hs, hd, edge_feats, wc, bp, smat, w2t, b2)
    pacc = _scatter()(mail2, dst, zrow)
    return _final_stage(pacc, h0, w3t, b3)


# speedup vs baseline: 5.9570x; 1.6619x over previous
import functools

import jax
import jax.numpy as jnp
from jax import lax
from jax.experimental import pallas as pl
from jax.experimental.pallas import tpu as pltpu
from jax.experimental.pallas import tpu_sc as plsc

N = 10000
E = 320000
IN_FEATS = 128
F = 16

CHUNK = 128
NCH = E // CHUNK
NW = 32
CPW = NCH // NW
CREM = NCH - CPW * NW
G = 8
NROW = N // 16

@functools.cache
def _sc_mesh():
    return plsc.VectorSubcoreMesh(core_axis_name="c", subcore_axis_name="s")


def _worker_range(w):
    nc = jnp.where(w < CREM, CPW + 1, CPW)
    cbase = w * CPW + jnp.minimum(w, CREM)
    return cbase, nc


def _e0(cbase, j):
    return pl.multiple_of((cbase + j) * CHUNK, CHUNK)


def _gather_body(h0_hbm, src_hbm, dst_hbm, hs_hbm, hd_hbm,
                 idx_s, idx_d, rows_s, rows_d, sem_i, sem_g, sem_w):
    cid = lax.axis_index("c")
    sid = lax.axis_index("s")
    w = sid * 2 + cid
    cbase, nc = _worker_range(w)

    def chunk_grp(c0, gsz):
        pend = []
        for b in range(gsz):
            e0 = _e0(cbase, c0 + b)
            pend.append(pltpu.async_copy(src_hbm.at[pl.ds(e0, CHUNK)], idx_s.at[b], sem_i))
            pend.append(pltpu.async_copy(dst_hbm.at[pl.ds(e0, CHUNK)], idx_d.at[b], sem_i))
        for d in pend:
            d.wait()
        pend = []
        for b in range(gsz):
            pend.append(pltpu.async_copy(h0_hbm.at[idx_s.at[b]], rows_s.at[b], sem_g))
            pend.append(pltpu.async_copy(h0_hbm.at[idx_d.at[b]], rows_d.at[b], sem_g))
        for d in pend:
            d.wait()
        pend = []
        for b in range(gsz):
            e0 = _e0(cbase, c0 + b)
            pend.append(pltpu.async_copy(rows_s.at[b], hs_hbm.at[pl.ds(e0, CHUNK)], sem_w))
            pend.append(pltpu.async_copy(rows_d.at[b], hd_hbm.at[pl.ds(e0, CHUNK)], sem_w))
        for d in pend:
            d.wait()

    def grp(g, carry):
        chunk_grp(g * G, G)
        return carry

    lax.fori_loop(0, nc // G, grp, 0)

    def tail(j, carry):
        chunk_grp(j, 1)
        return carry

    lax.fori_loop((nc // G) * G, nc, tail, 0)


@functools.cache
def _gather():
    return pl.kernel(
        _gather_body,
        out_type=[jax.ShapeDtypeStruct((E_PAD, F), jnp.float32),
                  jax.ShapeDtypeStruct((E_PAD, F), jnp.float32)],
        name="mpnn_gather",
        mesh=_sc_mesh(),
        compiler_params=pltpu.CompilerParams(use_tc_tiling_on_sc=False),
        scratch_types=[
            pltpu.VMEM((G, CHUNK), jnp.int32),
            pltpu.VMEM((G, CHUNK), jnp.int32),
            pltpu.VMEM((G, CHUNK, F), jnp.float32),
            pltpu.VMEM((G, CHUNK, F), jnp.float32),
            pltpu.SemaphoreType.DMA,
            pltpu.SemaphoreType.DMA,
            pltpu.SemaphoreType.DMA,
        ],
    )


def _scatter_body(mail2_hbm, dst_hbm, zrow_hbm, out_hbm,
                  idx_d, mail_v, zer_v, acc, sem_l, sem_s):
    cid = lax.axis_index("c")
    sid = lax.axis_index("s")
    w = sid * 2 + cid
    cbase, nc = _worker_range(w)

    pltpu.sync_copy(zrow_hbm, zer_v)
    pltpu.sync_copy(zer_v, acc.at[pl.ds(sid * NROW, NROW)])
    plsc.subcore_barrier()

    def chunk_grp(c0, gsz):
        pend = []
        for b in range(gsz):
            e0 = _e0(cbase, c0 + b)
            pend.append(pltpu.async_copy(dst_hbm.at[pl.ds(e0, CHUNK)], idx_d.at[b], sem_l))
            pend.append(pltpu.async_copy(mail2_hbm.at[pl.ds(e0, CHUNK)], mail_v.at[b], sem_l))
        for d in pend:
            d.wait()
        pend = []
        for b in range(gsz):
            pend.append(pltpu.async_copy(mail_v.at[b], acc.at[idx_d.at[b]], sem_s, add=True))
        for d in pend:
            d.wait()

    def grp(g, carry):
        chunk_grp(g * G, G)
        return carry

    lax.fori_loop(0, nc // G, grp, 0)

    def tail(j, carry):
        chunk_grp(j, 1)
        return carry

    lax.fori_loop((nc // G) * G, nc, tail, 0)

    plsc.subcore_barrier()
    pltpu.sync_copy(acc.at[pl.ds(sid * NROW, NROW)],
                    out_hbm.at[pl.ds(cid * N + sid * NROW, NROW)])


@functools.cache
def _scatter():
    return pl.kernel(
        _scatter_body,
        out_type=jax.ShapeDtypeStruct((2 * N, 2 * F), jnp.float32),
        name="mpnn_scatter",
        mesh=_sc_mesh(),
        compiler_params=pltpu.CompilerParams(use_tc_tiling_on_sc=False),
        scratch_types=[
            pltpu.VMEM((G, CHUNK), jnp.int32),
            pltpu.VMEM((G, CHUNK, 2 * F), jnp.float32),
            pltpu.VMEM((NROW, 2 * F), jnp.float32),
            pltpu.VMEM_SHARED((N, 2 * F), jnp.float32),
            pltpu.SemaphoreType.DMA,
            pltpu.SemaphoreType.DMA,
        ],
    )


def _node_body(x_ref, wnt_ref, bn_ref, out_ref):
    out_ref[...] = (jnp.dot(x_ref[...], wnt_ref[...],
                            preferred_element_type=jnp.float32) + bn_ref[...])


def _edge_body(hs_ref, hd_ref, ef_ref, wct_ref, st_ref, w2a_ref, out_ref):
    hst = jnp.transpose(hs_ref[...]).astype(jnp.bfloat16)
    hdt = jnp.transpose(hd_ref[...]).astype(jnp.bfloat16)
    eft = jnp.transpose(ef_ref[...]).astype(jnp.bfloat16)
    wct = wct_ref[...]
    st = st_ref[...]
    w2a = w2a_ref[...]
    onesr = jnp.ones((1, PBLK), jnp.bfloat16)
    onesf = jnp.ones((F, PBLK), jnp.float32)
    outs = []
    for k in range(8):
        r = F * k
        cat = jnp.concatenate([hst[r:r + F], hdt[r:r + F], eft[r:r + F],
                               onesr], axis=0)
        msgt = jnp.dot(wct, cat, preferred_element_type=jnp.float32)
        dt = jnp.dot(st, eft[r:r + F], preferred_element_type=jnp.float32)
        prod = jnp.concatenate([(dt * msgt).astype(jnp.bfloat16), onesr],
                               axis=0)
        mailt = jnp.dot(w2a, prod, preferred_element_type=jnp.float32)
        outs.append(mailt)
        outs.append(onesf)
    out_ref[...] = jnp.transpose(jnp.concatenate(outs, axis=0))


def _final_body(p0_ref, p1_ref, h0_ref, w3t_ref, b3_ref, out_ref):
    t = p0_ref[...] + p1_ref[...]
    s = t[:, 0:F]
    deg = t[:, F:F + 1]
    h0b = h0_ref[...]
    h = jnp.where(deg > 0.0, s / jnp.maximum(deg, 1.0), h0b)
    h = jnp.maximum(h, 0.0)
    w3t = w3t_ref[...]
    out_ref[...] = (h0b + b3_ref[...]
                    + jnp.dot(h0b, w3t[0:F], preferred_element_type=jnp.float32)
                    + jnp.dot(h, w3t[F:2 * F], preferred_element_type=jnp.float32))


NBLK = 2000
EBLK = 2000


def _node_linear(node_feats, wnt, bn):
    return pl.pallas_call(
        _node_body,
        grid=(N // NBLK,),
        in_specs=[pl.BlockSpec((NBLK, IN_FEATS), lambda i: (i, 0)),
                  pl.BlockSpec((IN_FEATS, F), lambda i: (0, 0)),
                  pl.BlockSpec((1, F), lambda i: (0, 0))],
        out_specs=pl.BlockSpec((NBLK, F), lambda i: (i, 0)),
        out_shape=jax.ShapeDtypeStruct((N, F), jnp.float32),
    )(node_feats, wnt, bn)


E8 = E // 8
PBLK = 512
E8P = 40960
E_PAD = E8P * 8


def _edge_stage(hsp, hdp, efp, wct, st, w2a):
    return pl.pallas_call(
        _edge_body,
        grid=(E8P // PBLK,),
        in_specs=[pl.BlockSpec((PBLK, 8 * F), lambda i: (i, 0)),
                  pl.BlockSpec((PBLK, 8 * F), lambda i: (i, 0)),
                  pl.BlockSpec((PBLK, 8 * F), lambda i: (i, 0)),
                  pl.BlockSpec((F * F, 3 * F + 1), lambda i: (0, 0)),
                  pl.BlockSpec((F * F, F), lambda i: (0, 0)),
                  pl.BlockSpec((F, F * F + 1), lambda i: (0, 0))],
        out_specs=pl.BlockSpec((PBLK, 16 * F), lambda i: (i, 0)),
        out_shape=jax.ShapeDtypeStruct((E8P, 16 * F), jnp.float32),
    )(hsp, hdp, efp, wct, st, w2a)


def _final_stage(pacc, h0, w3t, b3):
    nb = N // NBLK
    return pl.pallas_call(
        _final_body,
        grid=(nb,),
        in_specs=[pl.BlockSpec((NBLK, 2 * F), lambda i: (i, 0)),
                  pl.BlockSpec((NBLK, 2 * F), lambda i: (i + nb, 0)),
                  pl.BlockSpec((NBLK, F), lambda i: (i, 0)),
                  pl.BlockSpec((2 * F, F), lambda i: (0, 0)),
                  pl.BlockSpec((1, F), lambda i: (0, 0))],
        out_specs=pl.BlockSpec((NBLK, F), lambda i: (i, 0)),
        out_shape=jax.ShapeDtypeStruct((N, F), jnp.float32),
    )(pacc, pacc, h0, w3t, b3)


def kernel(node_feats, edge_feats, edge_index, W_node, b_node, W_edge, b_edge,
           W_m1, b_m1, W_m2, b_m2, W_m3, b_m3):
    a3 = W_m1[:, 2 * F:3 * F].T
    wc = jnp.concatenate([W_m1[:, 0:F].T, W_m1[:, F:2 * F].T,
                          W_edge.T @ a3], axis=0)
    bp = (b_m1 + b_edge @ a3).reshape(F * F, 1)
    wct = jnp.concatenate([wc.T, bp], axis=1)
    col = jax.lax.broadcasted_iota(jnp.int32, (F, F * F), 1) % F
    row = jax.lax.broadcasted_iota(jnp.int32, (F, F * F), 0)
    st = (col == row).astype(jnp.float32).T
    w2a = jnp.concatenate([W_m2, b_m2.reshape(F, 1)], axis=1)
    w3t = W_m3.T
    b3 = b_m3.reshape(1, F)
    bn = b_node.reshape(1, F)
    wnt = W_node.T

    src = edge_index[0]
    dst = edge_index[1]
    zrow = jnp.zeros((NROW, 2 * F), jnp.float32)

    h0 = _node_linear(node_feats, wnt, bn)
    hs, hd = _gather()(h0, src, dst)
    efp = jnp.pad(edge_feats.reshape(E8, 8 * F), ((0, E8P - E8), (0, 0)))
    mail2p = _edge_stage(hs.reshape(E8P, 8 * F), hd.reshape(E8P, 8 * F),
                         efp, wct.astype(jnp.bfloat16),
                         st.astype(jnp.bfloat16), w2a.astype(jnp.bfloat16))
    mail2 = mail2p.reshape(E_PAD, 2 * F)
    pacc = _scatter()(mail2, dst, zrow)
    return _final_stage(pacc, h0, w3t, b3)


# speedup vs baseline: 7.8994x; 1.3261x over previous
import functools

import jax
import jax.numpy as jnp
from jax import lax
from jax.experimental import pallas as pl
from jax.experimental.pallas import tpu as pltpu
from jax.experimental.pallas import tpu_sc as plsc

N = 10000
E = 320000
IN_FEATS = 128
F = 16

CHUNK = 128
NCH = E // CHUNK
NW = 32
CPW = NCH // NW
CREM = NCH - CPW * NW
G = 13
NROW = N // 16

E8 = E // 8
PBLK = 2048
E8P = 40960
E_PAD = E8P * 8


@functools.cache
def _sc_mesh():
    return plsc.VectorSubcoreMesh(core_axis_name="c", subcore_axis_name="s")


def _worker_range(w):
    nc = jnp.where(w < CREM, CPW + 1, CPW)
    cbase = w * CPW + jnp.minimum(w, CREM)
    return cbase, nc


def _e0(cbase, j):
    return pl.multiple_of((cbase + j) * CHUNK, CHUNK)


def _gather_body(h0_hbm, src_hbm, dst_hbm, hs_hbm, hd_hbm,
                 idx_s, idx_d, rows_s, rows_d, sem_i, sem_g, sem_w):
    cid = lax.axis_index("c")
    sid = lax.axis_index("s")
    w = sid * 2 + cid
    cbase, nc = _worker_range(w)

    def chunk_grp(c0, gsz):
        pend = []
        for b in range(gsz):
            e0 = _e0(cbase, c0 + b)
            pend.append(pltpu.async_copy(src_hbm.at[pl.ds(e0, CHUNK)], idx_s.at[b], sem_i))
            pend.append(pltpu.async_copy(dst_hbm.at[pl.ds(e0, CHUNK)], idx_d.at[b], sem_i))
        for d in pend:
            d.wait()
        pend = []
        for b in range(gsz):
            pend.append(pltpu.async_copy(h0_hbm.at[idx_s.at[b]], rows_s.at[b], sem_g))
            pend.append(pltpu.async_copy(h0_hbm.at[idx_d.at[b]], rows_d.at[b], sem_g))
        for d in pend:
            d.wait()
        pend = []
        for b in range(gsz):
            e0 = _e0(cbase, c0 + b)
            pend.append(pltpu.async_copy(rows_s.at[b], hs_hbm.at[pl.ds(e0, CHUNK)], sem_w))
            pend.append(pltpu.async_copy(rows_d.at[b], hd_hbm.at[pl.ds(e0, CHUNK)], sem_w))
        for d in pend:
            d.wait()

    def grp(g, carry):
        chunk_grp(g * G, G)
        return carry

    lax.fori_loop(0, nc // G, grp, 0)

    def tail(j, carry):
        chunk_grp(j, 1)
        return carry

    lax.fori_loop((nc // G) * G, nc, tail, 0)


@functools.cache
def _gather():
    return pl.kernel(
        _gather_body,
        out_type=[jax.ShapeDtypeStruct((E_PAD, F), jnp.float32),
                  jax.ShapeDtypeStruct((E_PAD, F), jnp.float32)],
        name="mpnn_gather",
        mesh=_sc_mesh(),
        compiler_params=pltpu.CompilerParams(use_tc_tiling_on_sc=False),
        scratch_types=[
            pltpu.VMEM((G, CHUNK), jnp.int32),
            pltpu.VMEM((G, CHUNK), jnp.int32),
            pltpu.VMEM((G, CHUNK, F), jnp.float32),
            pltpu.VMEM((G, CHUNK, F), jnp.float32),
            pltpu.SemaphoreType.DMA,
            pltpu.SemaphoreType.DMA,
            pltpu.SemaphoreType.DMA,
        ],
    )


def _scatter_body(mail2_hbm, dst_hbm, zrow_hbm, out_hbm,
                  idx_d, mail_v, zer_v, acc, sem_l, sem_s):
    cid = lax.axis_index("c")
    sid = lax.axis_index("s")
    w = sid * 2 + cid
    cbase, nc = _worker_range(w)

    pltpu.sync_copy(zrow_hbm, zer_v)
    pltpu.sync_copy(zer_v, acc.at[pl.ds(sid * NROW, NROW)])
    plsc.subcore_barrier()

    def chunk_grp(c0, gsz):
        pend = []
        for b in range(gsz):
            e0 = _e0(cbase, c0 + b)
            pend.append(pltpu.async_copy(dst_hbm.at[pl.ds(e0, CHUNK)], idx_d.at[b], sem_l))
            pend.append(pltpu.async_copy(mail2_hbm.at[pl.ds(e0, CHUNK)], mail_v.at[b], sem_l))
        for d in pend:
            d.wait()
        pend = []
        for b in range(gsz):
            pend.append(pltpu.async_copy(mail_v.at[b], acc.at[idx_d.at[b]], sem_s, add=True))
        for d in pend:
            d.wait()

    def grp(g, carry):
        chunk_grp(g * G, G)
        return carry

    lax.fori_loop(0, nc // G, grp, 0)

    def tail(j, carry):
        chunk_grp(j, 1)
        return carry

    lax.fori_loop((nc // G) * G, nc, tail, 0)

    plsc.subcore_barrier()
    pltpu.sync_copy(acc.at[pl.ds(sid * NROW, NROW)],
                    out_hbm.at[pl.ds(cid * N + sid * NROW, NROW)])


@functools.cache
def _scatter():
    return pl.kernel(
        _scatter_body,
        out_type=jax.ShapeDtypeStruct((2 * N, 2 * F), jnp.float32),
        name="mpnn_scatter",
        mesh=_sc_mesh(),
        compiler_params=pltpu.CompilerParams(use_tc_tiling_on_sc=False),
        scratch_types=[
            pltpu.VMEM((G, CHUNK), jnp.int32),
            pltpu.VMEM((G, CHUNK, 2 * F), jnp.float32),
            pltpu.VMEM((NROW, 2 * F), jnp.float32),
            pltpu.VMEM_SHARED((N, 2 * F), jnp.float32),
            pltpu.SemaphoreType.DMA,
            pltpu.SemaphoreType.DMA,
        ],
    )


def _node_body(x_ref, wnt_ref, bn_ref, out_ref):
    out_ref[...] = (jnp.dot(x_ref[...], wnt_ref[...],
                            preferred_element_type=jnp.float32) + bn_ref[...])


def _edge_body(hs_ref, hd_ref, ef_ref, wct_ref, w2a_ref, out_ref):
    eft32 = jnp.transpose(ef_ref[...])
    hst = jnp.transpose(hs_ref[...]).astype(jnp.bfloat16)
    hdt = jnp.transpose(hd_ref[...]).astype(jnp.bfloat16)
    eft = eft32.astype(jnp.bfloat16)
    wct = wct_ref[...]
    w2a = w2a_ref[...]
    onesr = jnp.ones((1, PBLK), jnp.bfloat16)
    onesf = jnp.ones((F, PBLK), jnp.float32)
    outs = []
    for k in range(8):
        r = F * k
        cat = jnp.concatenate([hst[r:r + F], hdt[r:r + F], eft[r:r + F],
                               onesr], axis=0)
        msgt = jnp.dot(wct, cat, preferred_element_type=jnp.float32)
        dt = jnp.tile(eft32[r:r + F], (F, 1))
        prod = jnp.concatenate([(dt * msgt).astype(jnp.bfloat16), onesr],
                               axis=0)
        mailt = jnp.dot(w2a, prod, preferred_element_type=jnp.float32)
        outs.append(mailt)
        outs.append(onesf)
    out_ref[...] = jnp.transpose(jnp.concatenate(outs, axis=0))


def _final_body(p0_ref, p1_ref, h0_ref, w3t_ref, b3_ref, out_ref):
    t = p0_ref[...] + p1_ref[...]
    s = t[:, 0:F]
    deg = t[:, F:F + 1]
    h0b = h0_ref[...]
    h = jnp.where(deg > 0.0, s / jnp.maximum(deg, 1.0), h0b)
    h = jnp.maximum(h, 0.0)
    w3t = w3t_ref[...]
    out_ref[...] = (h0b + b3_ref[...]
                    + jnp.dot(h0b, w3t[0:F], preferred_element_type=jnp.float32)
                    + jnp.dot(h, w3t[F:2 * F], preferred_element_type=jnp.float32))


NBLK = 2000


def _node_linear(node_feats, wnt, bn):
    return pl.pallas_call(
        _node_body,
        grid=(N // NBLK,),
        in_specs=[pl.BlockSpec((NBLK, IN_FEATS), lambda i: (i, 0)),
                  pl.BlockSpec((IN_FEATS, F), lambda i: (0, 0)),
                  pl.BlockSpec((1, F), lambda i: (0, 0))],
        out_specs=pl.BlockSpec((NBLK, F), lambda i: (i, 0)),
        out_shape=jax.ShapeDtypeStruct((N, F), jnp.float32),
    )(node_feats, wnt, bn)


def _edge_stage(hsp, hdp, efp, wct, w2a):
    return pl.pallas_call(
        _edge_body,
        grid=(E8P // PBLK,),
        in_specs=[pl.BlockSpec((PBLK, 8 * F), lambda i: (i, 0)),
                  pl.BlockSpec((PBLK, 8 * F), lambda i: (i, 0)),
                  pl.BlockSpec((PBLK, 8 * F), lambda i: (i, 0)),
                  pl.BlockSpec((F * F, 3 * F + 1), lambda i: (0, 0)),
                  pl.BlockSpec((F, F * F + 1), lambda i: (0, 0))],
        out_specs=pl.BlockSpec((PBLK, 16 * F), lambda i: (i, 0)),
        out_shape=jax.ShapeDtypeStruct((E8P, 16 * F), jnp.float32),
    )(hsp, hdp, efp, wct, w2a)


def _final_stage(pacc, h0, w3t, b3):
    nb = N // NBLK
    return pl.pallas_call(
        _final_body,
        grid=(nb,),
        in_specs=[pl.BlockSpec((NBLK, 2 * F), lambda i: (i, 0)),
                  pl.BlockSpec((NBLK, 2 * F), lambda i: (i + nb, 0)),
                  pl.BlockSpec((NBLK, F), lambda i: (i, 0)),
                  pl.BlockSpec((2 * F, F), lambda i: (0, 0)),
                  pl.BlockSpec((1, F), lambda i: (0, 0))],
        out_specs=pl.BlockSpec((NBLK, F), lambda i: (i, 0)),
        out_shape=jax.ShapeDtypeStruct((N, F), jnp.float32),
    )(pacc, pacc, h0, w3t, b3)


def kernel(node_feats, edge_feats, edge_index, W_node, b_node, W_edge, b_edge,
           W_m1, b_m1, W_m2, b_m2, W_m3, b_m3):
    efp = jnp.pad(edge_feats.reshape(E8, 8 * F), ((0, E8P - E8), (0, 0)))
    a3 = W_m1[:, 2 * F:3 * F].T
    wc = jnp.concatenate([W_m1[:, 0:F].T, W_m1[:, F:2 * F].T,
                          W_edge.T @ a3], axis=0)
    bp = (b_m1 + b_edge @ a3).reshape(F * F, 1)
    wct = jnp.concatenate([wc.T, bp], axis=1)
    w2a = jnp.concatenate([W_m2, b_m2.reshape(F, 1)], axis=1)
    w3t = W_m3.T
    b3 = b_m3.reshape(1, F)
    bn = b_node.reshape(1, F)
    wnt = W_node.T

    src = edge_index[0]
    dst = edge_index[1]
    zrow = jnp.zeros((NROW, 2 * F), jnp.float32)

    h0 = _node_linear(node_feats, wnt, bn)
    hs, hd = _gather()(h0, src, dst)
    mail2p = _edge_stage(hs.reshape(E8P, 8 * F), hd.reshape(E8P, 8 * F),
                         efp, wct.astype(jnp.bfloat16),
                         w2a.astype(jnp.bfloat16))
    pacc = _scatter()(mail2p.reshape(E_PAD, 2 * F), dst, zrow)
    return _final_stage(pacc, h0, w3t, b3)
